# baseline (speedup 1.0000x reference)
import jax
import jax.numpy as jnp
from jax import lax
from jax.experimental import pallas as pl
from jax.experimental.pallas import tpu as pltpu
from jax.experimental.pallas import tpu_sc as plsc

VOCAB = 30522
HIDDEN = 768
MAXPOS = 512
B = 128
L = 512
EPS = 1e-12

NC, NS = 2, 16
NW = NC * NS
TOK = B * L
TPW = TOK // NW
CH = 64
NCHUNK = TPW // CH

LB = 512
NJ = L // LB
BB = 8


VB = 512


HW = HIDDEN // 2


def _conv_body(w_ref, o_ref):
    wb = w_ref[...].astype(jnp.bfloat16).reshape(2 * VB, HW)
    o_ref[...] = pltpu.bitcast(wb, jnp.int32)


_conv_call = pl.pallas_call(
    _conv_body,
    grid=((VOCAB + VB - 1) // VB,),
    in_specs=[pl.BlockSpec((VB, HIDDEN), lambda v: (v, 0))],
    out_specs=pl.BlockSpec((VB, HW), lambda v: (v, 0)),
    out_shape=jax.ShapeDtypeStruct((VOCAB, HW), jnp.int32),
    compiler_params=pltpu.CompilerParams(
        dimension_semantics=("arbitrary",)),
)


def _gather_body(ids_hbm, word_hbm, y_hbm,
                 ids_v, buf_a, buf_b, sem_a, sem_b, sem_oa, sem_ob):
    wid = lax.axis_index("s") * NC + lax.axis_index("c")
    base0 = wid * TPW
    pltpu.sync_copy(ids_hbm.at[pl.ds(base0, TPW)], ids_v)

    slots = ((buf_a, sem_a, sem_oa), (buf_b, sem_b, sem_ob))

    def issue(c, buf, sem_g):
        pltpu.async_copy(word_hbm.at[ids_v.at[pl.ds(c * CH, CH)]], buf, sem_g)

    for s in range(2):
        issue(s, slots[s][0], slots[s][1])

    def half(h, carry):
        for s in range(2):
            buf, sem_g, sem_o = slots[s]
            c = 2 * h + s
            pltpu.make_async_copy(word_hbm.at[pl.ds(0, CH)], buf,
                                  sem_g).wait()
            pltpu.async_copy(buf, y_hbm.at[pl.ds(base0 + c * CH, CH)], sem_o)
            pltpu.make_async_copy(buf, y_hbm.at[pl.ds(0, CH)], sem_o).wait()
            cn = jnp.minimum(c + 2, NCHUNK - 1)
            issue(cn, buf, sem_g)
        return carry

    lax.fori_loop(0, NCHUNK // 2, half, 0)
    for s in range(2):
        buf, sem_g, sem_o = slots[s]
        pltpu.make_async_copy(word_hbm.at[pl.ds(0, CH)], buf, sem_g).wait()


_gather_call = pl.kernel(
    _gather_body,
    out_type=jax.ShapeDtypeStruct((TOK, HW), jnp.int32),
    mesh=plsc.VectorSubcoreMesh(core_axis_name="c", subcore_axis_name="s",
                                num_cores=NC, num_subcores=NS),
    scratch_types=[
        pltpu.VMEM((TPW,), jnp.int32),
        pltpu.VMEM((CH, HW), jnp.int32),
        pltpu.VMEM((CH, HW), jnp.int32),
        pltpu.SemaphoreType.DMA,
        pltpu.SemaphoreType.DMA,
        pltpu.SemaphoreType.DMA,
        pltpu.SemaphoreType.DMA,
    ],
    compiler_params=pltpu.CompilerParams(needs_layout_passes=False),
)


def _ln_body(y_ref, tt_ref, pos_ref, type_ref, gamma_ref, beta_ref, out_ref):
    for bb in range(BB):
        yb = pltpu.bitcast(y_ref[bb], jnp.bfloat16)
        x = yb.reshape(LB, HIDDEN).astype(jnp.float32) + pos_ref[...]
        ttf = tt_ref[bb, 0, :]
        t0 = type_ref[0, :]
        dt = type_ref[1, :] - t0
        x = x + t0[None, :] + ttf[:, None] * dt[None, :]
        mean = jnp.mean(x, axis=-1, keepdims=True)
        var = jnp.mean(jnp.square(x - mean), axis=-1, keepdims=True)
        x = (x - mean) * lax.rsqrt(var + EPS)
        out_ref[bb] = x * gamma_ref[...] + beta_ref[...]


_ln_call = pl.pallas_call(
    _ln_body,
    grid=(B // BB,),
    in_specs=[
        pl.BlockSpec((BB, LB, HW), lambda b: (b, 0, 0)),
        pl.BlockSpec((BB, 1, L), lambda b: (b, 0, 0)),
        pl.BlockSpec((LB, HIDDEN), lambda b: (0, 0)),
        pl.BlockSpec((2, HIDDEN), lambda b: (0, 0)),
        pl.BlockSpec((HIDDEN,), lambda b: (0,)),
        pl.BlockSpec((HIDDEN,), lambda b: (0,)),
    ],
    out_specs=pl.BlockSpec((BB, LB, HIDDEN), lambda b: (b, 0, 0)),
    out_shape=jax.ShapeDtypeStruct((B, L, HIDDEN), jnp.float32),
    compiler_params=pltpu.CompilerParams(
        dimension_semantics=("arbitrary",)),
)


def kernel(input_ids, token_type_ids, word_emb, pos_emb, type_emb,
           ln_gamma, ln_beta):
    ids = input_ids.reshape(-1).astype(jnp.int32)
    ttf = token_type_ids.astype(jnp.float32).reshape(B, 1, L)
    word_bf = _conv_call(word_emb)
    y = _gather_call(ids, word_bf)
    y = y.reshape(B, L, HW)
    return _ln_call(y, ttf, pos_emb, type_emb, ln_gamma, ln_beta)


# speedup vs baseline: 1.3584x; 1.3584x over previous
import jax
import jax.numpy as jnp
from jax import lax
from jax.experimental import pallas as pl
from jax.experimental.pallas import tpu as pltpu
from jax.experimental.pallas import tpu_sc as plsc

VOCAB = 30522
HIDDEN = 768
MAXPOS = 512
B = 128
L = 512
EPS = 1e-12

NC, NS = 2, 16
NW = NC * NS
TOK = B * L
TPW = TOK // NW
CH = 64
NCHUNK = TPW // CH

LB = 512
NJ = L // LB
BB = 8


VB = 512


HW = HIDDEN // 2


def _conv_body(w_ref, o_ref):
    u = pltpu.bitcast(w_ref[...], jnp.int32)
    r = (u + 0x7FFF + ((u >> 16) & 1)) >> 16
    o_ref[...] = (r[:, :HW] & 0xFFFF) | (r[:, HW:] << 16)


_conv_call = pl.pallas_call(
    _conv_body,
    grid=((VOCAB + VB - 1) // VB,),
    in_specs=[pl.BlockSpec((VB, HIDDEN), lambda v: (v, 0))],
    out_specs=pl.BlockSpec((VB, HW), lambda v: (v, 0)),
    out_shape=jax.ShapeDtypeStruct((VOCAB, HW), jnp.int32),
    compiler_params=pltpu.CompilerParams(
        dimension_semantics=("arbitrary",)),
)


def _gather_body(ids_hbm, word_hbm, y_hbm,
                 ids_v, buf_a, buf_b, sem_a, sem_b, sem_oa, sem_ob):
    wid = lax.axis_index("s") * NC + lax.axis_index("c")
    base0 = wid * TPW
    pltpu.sync_copy(ids_hbm.at[pl.ds(base0, TPW)], ids_v)

    slots = ((buf_a, sem_a, sem_oa), (buf_b, sem_b, sem_ob))

    def issue(c, buf, sem_g):
        pltpu.async_copy(word_hbm.at[ids_v.at[pl.ds(c * CH, CH)]], buf, sem_g)

    for s in range(2):
        issue(s, slots[s][0], slots[s][1])

    def half(h, carry):
        for s in range(2):
            buf, sem_g, sem_o = slots[s]
            c = 2 * h + s
            pltpu.make_async_copy(word_hbm.at[pl.ds(0, CH)], buf,
                                  sem_g).wait()
            pltpu.async_copy(buf, y_hbm.at[pl.ds(base0 + c * CH, CH)], sem_o)
            pltpu.make_async_copy(buf, y_hbm.at[pl.ds(0, CH)], sem_o).wait()
            cn = jnp.minimum(c + 2, NCHUNK - 1)
            issue(cn, buf, sem_g)
        return carry

    lax.fori_loop(0, NCHUNK // 2, half, 0)
    for s in range(2):
        buf, sem_g, sem_o = slots[s]
        pltpu.make_async_copy(word_hbm.at[pl.ds(0, CH)], buf, sem_g).wait()


_gather_call = pl.kernel(
    _gather_body,
    out_type=jax.ShapeDtypeStruct((TOK, HW), jnp.int32),
    mesh=plsc.VectorSubcoreMesh(core_axis_name="c", subcore_axis_name="s",
                                num_cores=NC, num_subcores=NS),
    scratch_types=[
        pltpu.VMEM((TPW,), jnp.int32),
        pltpu.VMEM((CH, HW), jnp.int32),
        pltpu.VMEM((CH, HW), jnp.int32),
        pltpu.SemaphoreType.DMA,
        pltpu.SemaphoreType.DMA,
        pltpu.SemaphoreType.DMA,
        pltpu.SemaphoreType.DMA,
    ],
    compiler_params=pltpu.CompilerParams(needs_layout_passes=False),
)


def _ln_body(y_ref, tt_ref, pos_ref, type_ref, gamma_ref, beta_ref, out_ref):
    t0 = type_ref[0, :]
    dt = type_ref[1, :] - t0
    gam = gamma_ref[...]
    bet = beta_ref[...]
    pos = pos_ref[...]
    for bb in range(BB):
        y = y_ref[bb]
        xf = pltpu.bitcast(y << 16, jnp.float32)
        xs = pltpu.bitcast(y & jnp.int32(-65536), jnp.float32)
        ttf = tt_ref[bb, 0, :]
        xf = xf + pos[:, :HW] + (t0[None, :HW]
                                 + ttf[:, None] * dt[None, :HW])
        xs = xs + pos[:, HW:] + (t0[None, HW:]
                                 + ttf[:, None] * dt[None, HW:])
        s = (jnp.sum(xf, axis=-1, keepdims=True)
             + jnp.sum(xs, axis=-1, keepdims=True))
        q = (jnp.sum(xf * xf, axis=-1, keepdims=True)
             + jnp.sum(xs * xs, axis=-1, keepdims=True))
        mean = s * (1.0 / HIDDEN)
        var = q * (1.0 / HIDDEN) - mean * mean
        inv = lax.rsqrt(var + EPS)
        out_ref[bb, :, :HW] = ((xf - mean) * inv * gam[None, :HW]
                               + bet[None, :HW])
        out_ref[bb, :, HW:] = ((xs - mean) * inv * gam[None, HW:]
                               + bet[None, HW:])


_ln_call = pl.pallas_call(
    _ln_body,
    grid=(B // BB,),
    in_specs=[
        pl.BlockSpec((BB, LB, HW), lambda b: (b, 0, 0)),
        pl.BlockSpec((BB, 1, L), lambda b: (b, 0, 0)),
        pl.BlockSpec((LB, HIDDEN), lambda b: (0, 0)),
        pl.BlockSpec((2, HIDDEN), lambda b: (0, 0)),
        pl.BlockSpec((HIDDEN,), lambda b: (0,)),
        pl.BlockSpec((HIDDEN,), lambda b: (0,)),
    ],
    out_specs=pl.BlockSpec((BB, LB, HIDDEN), lambda b: (b, 0, 0)),
    out_shape=jax.ShapeDtypeStruct((B, L, HIDDEN), jnp.float32),
    compiler_params=pltpu.CompilerParams(
        dimension_semantics=("arbitrary",)),
)


def kernel(input_ids, token_type_ids, word_emb, pos_emb, type_emb,
           ln_gamma, ln_beta):
    ids = input_ids.reshape(-1).astype(jnp.int32)
    ttf = token_type_ids.astype(jnp.float32).reshape(B, 1, L)
    word_bf = _conv_call(word_emb)
    y = _gather_call(ids, word_bf)
    y = y.reshape(B, L, HW)
    return _ln_call(y, ttf, pos_emb, type_emb, ln_gamma, ln_beta)
